# baseline (speedup 1.0000x reference)
import functools

import jax
import jax.numpy as jnp
from jax import lax
from jax.experimental import pallas as pl
from jax.experimental.pallas import tpu as pltpu

_MM_DTYPE = jnp.float32


def _cell(gi, h, whh, bhh):
    H = h.shape[-1]
    gh = jnp.dot(h.astype(whh.dtype), whh,
                 preferred_element_type=jnp.float32) + bhh
    r = jax.nn.sigmoid(gi[:, 0:H] + gh[:, 0:H])
    z = jax.nn.sigmoid(gi[:, H:2 * H] + gh[:, H:2 * H])
    n = jnp.tanh(gi[:, 2 * H:3 * H] + r * gh[:, 2 * H:3 * H])
    return n + z * (h - n)


def _dir_chunk_kernel(seq_ref, x_ref, wih_ref, bih_ref, whh_ref, bhh_ref,
                      out_ref, h_ref, gi_ref, *, t_chunk, bp, unroll):
    d = pl.program_id(0)
    i = pl.program_id(1)
    n_chunks = pl.num_programs(1)

    @pl.when(i == 0)
    def _():
        h_ref[...] = jnp.zeros_like(h_ref)

    gi_ref[...] = (jnp.dot(x_ref[...], wih_ref[0],
                           preferred_element_type=jnp.float32)
                   + bih_ref[0])

    seq = seq_ref[...]
    chunk = (1 - d) * i + d * (n_chunks - 1 - i)
    t0 = chunk * t_chunk
    whh = whh_ref[0]
    bhh = bhh_ref[0]

    def body(j, h):
        s = jnp.where(d == 0, j, t_chunk - 1 - j)
        row = pl.multiple_of(s * bp, bp)
        m = (seq > (t0 + s)).astype(jnp.float32)
        h_new = _cell(gi_ref[pl.ds(row, bp), :], h, whh, bhh)
        h_next = h + m * (h_new - h)
        out_ref[0, pl.ds(row, bp), :] = m * h_next
        return h_next

    h_ref[...] = lax.fori_loop(0, t_chunk, body, h_ref[...], unroll=unroll)


def _bigru(x_bth, seq_lengths, wih, bih, whh, bhh, *,
           t_chunk=8, unroll=8, sublane=8):
    B, T, H = x_bth.shape
    Bp = ((B + sublane - 1) // sublane) * sublane
    t_chunk = max(1, min(t_chunk, T))
    n_chunks = pl.cdiv(T, t_chunk)
    Tp = n_chunks * t_chunk
    blk = t_chunk * Bp

    x_tbh = jnp.transpose(x_bth, (1, 0, 2))
    seq_i32 = seq_lengths.astype(jnp.int32)
    if Tp != T or Bp != B:
        x_tbh = jnp.pad(x_tbh, ((0, Tp - T), (0, Bp - B), (0, 0)))
    if Bp != B:
        seq_i32 = jnp.pad(seq_i32, (0, Bp - B))
    x2d = x_tbh.reshape(Tp * Bp, H).astype(_MM_DTYPE)
    seq2d = seq_i32.reshape(Bp, 1)

    const = lambda d, i: (0, 0)
    per_dir = lambda d, i: (d, 0, 0)
    x_spec = pl.BlockSpec(
        (blk, H), lambda d, i: ((1 - d) * i + d * (n_chunks - 1 - i), 0))
    o_spec = pl.BlockSpec(
        (1, blk, H), lambda d, i: (d, (1 - d) * i + d * (n_chunks - 1 - i), 0))

    kernel_fn = functools.partial(_dir_chunk_kernel, t_chunk=t_chunk,
                                  bp=Bp, unroll=min(unroll, t_chunk))

    mm_bytes = jnp.dtype(_MM_DTYPE).itemsize
    vmem_bytes = int(min(
        2 * (blk * H * mm_bytes + blk * H * 4)
        + blk * 3 * H * 4
        + 2 * H * 3 * H * mm_bytes
        + Bp * H * 4 + (8 << 20),
        100 << 20))

    out = pl.pallas_call(
        kernel_fn,
        out_shape=jax.ShapeDtypeStruct((2, Tp * Bp, H), jnp.float32),
        grid=(2, n_chunks),
        in_specs=[
            pl.BlockSpec(seq2d.shape, const),
            x_spec,
            pl.BlockSpec((1,) + wih.shape[1:], per_dir),
            pl.BlockSpec((1,) + bih.shape[1:], per_dir),
            pl.BlockSpec((1,) + whh.shape[1:], per_dir),
            pl.BlockSpec((1,) + bhh.shape[1:], per_dir),
        ],
        out_specs=o_spec,
        scratch_shapes=[
            pltpu.VMEM((Bp, H), jnp.float32),
            pltpu.VMEM((blk, 3 * H), jnp.float32),
        ],
        compiler_params=pltpu.CompilerParams(
            dimension_semantics=("parallel", "arbitrary"),
            vmem_limit_bytes=vmem_bytes),
    )(seq2d, x2d, wih, bih, whh, bhh)

    out = out.reshape(2, Tp, Bp, H)
    return jnp.concatenate([out[0, :T, :B], out[1, :T, :B]], axis=-1)


def kernel(x_bth, seq_lengths, w_ih_f, w_hh_f, b_ih_f, b_hh_f,
           w_ih_b, w_hh_b, b_ih_b, b_hh_b, embedding, fc_w, fc_b):
    mm = _MM_DTYPE
    wih = jnp.stack([w_ih_f, w_ih_b]).astype(mm)
    whh = jnp.stack([w_hh_f, w_hh_b]).astype(mm)
    bih = jnp.stack([b_ih_f, b_ih_b])
    bhh = jnp.stack([b_hh_f, b_hh_b])
    return _bigru(x_bth, seq_lengths, wih, bih, whh, bhh)


# speedup vs baseline: 1.6803x; 1.6803x over previous
import functools

import jax
import jax.numpy as jnp
from jax import lax
from jax.experimental import pallas as pl
from jax.experimental.pallas import tpu as pltpu


def _cell(gi, h, whh, bhh):
    H = h.shape[-1]
    gh = jnp.dot(h.astype(whh.dtype), whh,
                 preferred_element_type=jnp.float32) + bhh
    rz = jax.nn.sigmoid(gi[:, 0:2 * H] + gh[:, 0:2 * H])
    r = rz[:, 0:H]
    z = rz[:, H:2 * H]
    n = jnp.tanh(gi[:, 2 * H:3 * H] + r * gh[:, 2 * H:3 * H])
    return n + z * (h - n)


def _dir_chunk_kernel(seq_ref, x_ref, wih_ref, bih_ref, whh_ref, bhh_ref,
                      out_ref, h_ref, *, t_chunk, unroll):
    d = pl.program_id(0)
    i = pl.program_id(1)
    n_chunks = pl.num_programs(1)

    @pl.when(i == 0)
    def _():
        h_ref[...] = jnp.zeros_like(h_ref)

    seq = seq_ref[...]
    H = h_ref.shape[-1]
    wih = wih_ref[0]
    bih = bih_ref[0]
    whh = whh_ref[0]
    bhh = bhh_ref[0]

    def step(s, t, h):
        xj = x_ref[:, pl.ds(pl.multiple_of(s * H, H), H)]
        gi = jnp.dot(xj, wih, preferred_element_type=jnp.float32) + bih
        h_new = _cell(gi, h, whh, bhh)
        m = seq > t
        out_ref[s] = jnp.where(m, h_new, 0.0)
        return jnp.where(m, h_new, h)

    @pl.when(d == 0)
    def _():
        t0 = i * t_chunk
        h_ref[...] = lax.fori_loop(
            0, t_chunk, lambda j, h: step(j, t0 + j, h), h_ref[...],
            unroll=unroll)

    @pl.when(d == 1)
    def _():
        t0 = (n_chunks - 1 - i) * t_chunk
        h_ref[...] = lax.fori_loop(
            0, t_chunk, lambda j, h: step(t_chunk - 1 - j,
                                          t0 + t_chunk - 1 - j, h),
            h_ref[...], unroll=unroll)


def _bigru(x_bth, seq_lengths, wih, bih, whh, bhh, *, t_chunk=16, unroll=16):
    B, T, H = x_bth.shape
    t_chunk = max(1, min(t_chunk, T))
    if T % t_chunk or B % 8:
        t_chunk = 8 if T % 8 == 0 else 1
    n_chunks = T // t_chunk

    x2v = x_bth.reshape(B, T * H)
    seq2d = seq_lengths.astype(jnp.int32).reshape(B, 1)

    const = lambda d, i: (0, 0)
    per_dir = lambda d, i: (d, 0, 0)
    chunk_ix = lambda d, i: (1 - d) * i + d * (n_chunks - 1 - i)
    x_spec = pl.BlockSpec((B, t_chunk * H), lambda d, i: (0, chunk_ix(d, i)))
    o_spec = pl.BlockSpec((t_chunk, B, H),
                          lambda d, i: (chunk_ix(d, i), 0, d))

    kernel_fn = functools.partial(_dir_chunk_kernel, t_chunk=t_chunk,
                                  unroll=min(unroll, t_chunk))

    blk_bytes = t_chunk * B * H * 4
    vmem_bytes = int(min(2 * 2 * blk_bytes
                         + 2 * 2 * H * 3 * H * 4
                         + B * H * 4 + (16 << 20),
                         100 << 20))

    out = pl.pallas_call(
        kernel_fn,
        out_shape=jax.ShapeDtypeStruct((T, B, 2 * H), jnp.float32),
        grid=(2, n_chunks),
        in_specs=[
            pl.BlockSpec(seq2d.shape, const),
            x_spec,
            pl.BlockSpec((1,) + wih.shape[1:], per_dir),
            pl.BlockSpec((1,) + bih.shape[1:], per_dir),
            pl.BlockSpec((1,) + whh.shape[1:], per_dir),
            pl.BlockSpec((1,) + bhh.shape[1:], per_dir),
        ],
        out_specs=o_spec,
        scratch_shapes=[
            pltpu.VMEM((B, H), jnp.float32),
        ],
        compiler_params=pltpu.CompilerParams(
            dimension_semantics=("parallel", "arbitrary"),
            vmem_limit_bytes=vmem_bytes),
    )(seq2d, x2v, wih, bih, whh, bhh)

    return out


def kernel(x_bth, seq_lengths, w_ih_f, w_hh_f, b_ih_f, b_hh_f,
           w_ih_b, w_hh_b, b_ih_b, b_hh_b, embedding, fc_w, fc_b):
    wih = jnp.stack([w_ih_f, w_ih_b])
    whh = jnp.stack([w_hh_f, w_hh_b])
    bih = jnp.stack([b_ih_f, b_ih_b])
    bhh = jnp.stack([b_hh_f, b_hh_b])
    return _bigru(x_bth, seq_lengths, wih, bih, whh, bhh)


# speedup vs baseline: 1.8348x; 1.0919x over previous
import functools

import jax
import jax.numpy as jnp
from jax import lax
from jax.experimental import pallas as pl
from jax.experimental.pallas import tpu as pltpu


def _dir_chunk_kernel(seq_ref, x_ref, wih_ref, whh_ref,
                      brz_ref, bihn_ref, bhhn_ref,
                      out_ref, h_ref, *, t_chunk, unroll):
    d = pl.program_id(0)
    i = pl.program_id(1)
    n_chunks = pl.num_programs(1)

    @pl.when(i == 0)
    def _():
        h_ref[...] = jnp.zeros_like(h_ref)

    seq = seq_ref[...]
    H = h_ref.shape[-1]
    wih = wih_ref[0]
    whh = whh_ref[0]
    brz = brz_ref[0]
    bihn = bihn_ref[0]
    bhhn = bhhn_ref[0]

    def step(s, t, h):
        xj = x_ref[:, pl.ds(pl.multiple_of(s * H, H), H)]
        gi = jnp.dot(xj.astype(jnp.bfloat16), wih,
                     preferred_element_type=jnp.float32)
        gh = jnp.dot(h.astype(jnp.bfloat16), whh,
                     preferred_element_type=jnp.float32)
        rz = jnp.tanh(gi[:, 0:2 * H] + gh[:, 0:2 * H] + brz) * 0.5 + 0.5
        r = rz[:, 0:H]
        z = rz[:, H:2 * H]
        n = jnp.tanh(gi[:, 2 * H:] + bihn + r * (gh[:, 2 * H:] + bhhn))
        h_new = n + z * (h - n)
        m = seq > t
        out_ref[s] = jnp.where(m, h_new, 0.0)
        return jnp.where(m, h_new, h)

    @pl.when(d == 0)
    def _():
        t0 = i * t_chunk
        h_ref[...] = lax.fori_loop(
            0, t_chunk, lambda j, h: step(j, t0 + j, h), h_ref[...],
            unroll=unroll)

    @pl.when(d == 1)
    def _():
        t0 = (n_chunks - 1 - i) * t_chunk
        h_ref[...] = lax.fori_loop(
            0, t_chunk, lambda j, h: step(t_chunk - 1 - j,
                                          t0 + t_chunk - 1 - j, h),
            h_ref[...], unroll=unroll)


def _bigru(x_bth, seq_lengths, wih, whh, brz, bihn, bhhn, *,
           t_chunk=16, unroll=16):
    B, T, H = x_bth.shape
    t_chunk = max(1, min(t_chunk, T))
    if T % t_chunk or B % 8:
        t_chunk = 8 if T % 8 == 0 else 1
    n_chunks = T // t_chunk

    x2v = x_bth.reshape(B, T * H)
    seq2d = seq_lengths.astype(jnp.int32).reshape(B, 1)

    const = lambda d, i: (0, 0)
    per_dir = lambda d, i: (d, 0, 0)
    chunk_ix = lambda d, i: (1 - d) * i + d * (n_chunks - 1 - i)
    x_spec = pl.BlockSpec((B, t_chunk * H), lambda d, i: (0, chunk_ix(d, i)))
    o_spec = pl.BlockSpec((t_chunk, B, H),
                          lambda d, i: (chunk_ix(d, i), 0, d))

    kernel_fn = functools.partial(_dir_chunk_kernel, t_chunk=t_chunk,
                                  unroll=min(unroll, t_chunk))

    blk_bytes = t_chunk * B * H * 4
    vmem_bytes = int(min(2 * 2 * blk_bytes
                         + 2 * 2 * H * 3 * H * 4
                         + B * H * 4 + (16 << 20),
                         100 << 20))

    out = pl.pallas_call(
        kernel_fn,
        out_shape=jax.ShapeDtypeStruct((T, B, 2 * H), jnp.float32),
        grid=(2, n_chunks),
        in_specs=[
            pl.BlockSpec(seq2d.shape, const),
            x_spec,
            pl.BlockSpec((1,) + wih.shape[1:], per_dir),
            pl.BlockSpec((1,) + whh.shape[1:], per_dir),
            pl.BlockSpec((1,) + brz.shape[1:], per_dir),
            pl.BlockSpec((1,) + bihn.shape[1:], per_dir),
            pl.BlockSpec((1,) + bhhn.shape[1:], per_dir),
        ],
        out_specs=o_spec,
        scratch_shapes=[
            pltpu.VMEM((B, H), jnp.float32),
        ],
        compiler_params=pltpu.CompilerParams(
            dimension_semantics=("arbitrary", "arbitrary"),
            vmem_limit_bytes=vmem_bytes),
    )(seq2d, x2v, wih, whh, brz, bihn, bhhn)

    return out


def kernel(x_bth, seq_lengths, w_ih_f, w_hh_f, b_ih_f, b_hh_f,
           w_ih_b, w_hh_b, b_ih_b, b_hh_b, embedding, fc_w, fc_b):
    H = x_bth.shape[-1]
    scale = jnp.concatenate([jnp.full((1, 2 * H), 0.5, jnp.float32),
                             jnp.ones((1, H), jnp.float32)], axis=-1)
    wih = (jnp.stack([w_ih_f, w_ih_b]) * scale).astype(jnp.bfloat16)
    whh = (jnp.stack([w_hh_f, w_hh_b]) * scale).astype(jnp.bfloat16)
    brz = (jnp.stack([b_ih_f + b_hh_f, b_ih_b + b_hh_b])[:, :, :2 * H]
           * jnp.float32(0.5))
    bihn = jnp.stack([b_ih_f, b_ih_b])[:, :, 2 * H:]
    bhhn = jnp.stack([b_hh_f, b_hh_b])[:, :, 2 * H:]
    return _bigru(x_bth, seq_lengths, wih, whh, brz, bihn, bhhn)


# speedup vs baseline: 2.6811x; 1.4613x over previous
import functools

import jax
import jax.numpy as jnp
from jax import lax
from jax.experimental import pallas as pl
from jax.experimental.pallas import tpu as pltpu


def _gates(xj, h, wih, whh, brz, bihn, bhhn):
    H = h.shape[-1]
    gi = jnp.dot(xj.astype(jnp.bfloat16), wih,
                 preferred_element_type=jnp.float32)
    gh = jnp.dot(h.astype(jnp.bfloat16), whh,
                 preferred_element_type=jnp.float32)
    rz = jnp.tanh(gi[:, 0:2 * H] + gh[:, 0:2 * H] + brz) * 0.5 + 0.5
    r = rz[:, 0:H]
    z = rz[:, H:2 * H]
    n = jnp.tanh(gi[:, 2 * H:] + bihn + r * (gh[:, 2 * H:] + bhhn))
    return n + z * (h - n)


def _bigru_chunk_kernel(seq_ref, xf_ref, xb_ref,
                        wih_f_ref, whh_f_ref, brz_f_ref, bihn_f_ref,
                        bhhn_f_ref, wih_b_ref, whh_b_ref, brz_b_ref,
                        bihn_b_ref, bhhn_b_ref,
                        out_ref,
                        hf_ref, hb_ref, of_ref, ob_ref, sf_sem, sb_sem,
                        *, t_chunk, unroll):
    i = pl.program_id(0)
    n_chunks = pl.num_programs(0)
    B, H = hf_ref.shape
    p = lax.rem(i, 2)

    @pl.when(i == 0)
    def _():
        hf_ref[...] = jnp.zeros_like(hf_ref)
        hb_ref[...] = jnp.zeros_like(hb_ref)

    def copies(slot, chunk_f, chunk_b):
        cf = pltpu.make_async_copy(
            of_ref.at[slot],
            out_ref.at[pl.ds(chunk_f * t_chunk, t_chunk), :, pl.ds(0, H)],
            sf_sem.at[slot])
        cb = pltpu.make_async_copy(
            ob_ref.at[slot],
            out_ref.at[pl.ds(chunk_b * t_chunk, t_chunk), :, pl.ds(H, H)],
            sb_sem.at[slot])
        return cf, cb

    @pl.when(i >= 2)
    def _():
        cf, cb = copies(p, i - 2, n_chunks + 1 - i)
        cf.wait()
        cb.wait()

    seq = seq_ref[...]
    wih_f = wih_f_ref[...]
    whh_f = whh_f_ref[...]
    wih_b = wih_b_ref[...]
    whh_b = whh_b_ref[...]
    brz_f = brz_f_ref[...]
    bihn_f = bihn_f_ref[...]
    bhhn_f = bhhn_f_ref[...]
    brz_b = brz_b_ref[...]
    bihn_b = bihn_b_ref[...]
    bhhn_b = bhhn_b_ref[...]
    t0f = i * t_chunk
    t0b = (n_chunks - 1 - i) * t_chunk

    def body(j, carry):
        hf, hb = carry
        jr = t_chunk - 1 - j
        xf = xf_ref[:, pl.ds(pl.multiple_of(j * H, H), H)]
        xb = xb_ref[:, pl.ds(pl.multiple_of(jr * H, H), H)]
        hf_new = _gates(xf, hf, wih_f, whh_f, brz_f, bihn_f, bhhn_f)
        hb_new = _gates(xb, hb, wih_b, whh_b, brz_b, bihn_b, bhhn_b)
        mf = seq > (t0f + j)
        mb = seq > (t0b + jr)
        of_ref[p, j] = jnp.where(mf, hf_new, 0.0)
        ob_ref[p, jr] = jnp.where(mb, hb_new, 0.0)
        return (jnp.where(mf, hf_new, hf),
                jnp.where(mb, hb_new, hb))

    hf, hb = lax.fori_loop(0, t_chunk, body, (hf_ref[...], hb_ref[...]),
                           unroll=unroll)
    hf_ref[...] = hf
    hb_ref[...] = hb

    cf, cb = copies(p, i, n_chunks - 1 - i)
    cf.start()
    cb.start()

    if n_chunks > 1:
        @pl.when(i == n_chunks - 1)
        def _():
            cf2, cb2 = copies(1 - p, i - 1, n_chunks - i)
            cf2.wait()
            cb2.wait()

    @pl.when(i == n_chunks - 1)
    def _():
        cf3, cb3 = copies(p, i, n_chunks - 1 - i)
        cf3.wait()
        cb3.wait()


def _bigru(x_bth, seq_lengths, wih_f, whh_f, brz_f, bihn_f, bhhn_f,
           wih_b, whh_b, brz_b, bihn_b, bhhn_b, *, t_chunk=16, unroll=16):
    B, T, H = x_bth.shape
    t_chunk = max(1, min(t_chunk, T))
    if T % t_chunk or B % 8:
        t_chunk = 8 if T % 8 == 0 else 1
    n_chunks = T // t_chunk

    x2v = x_bth.reshape(B, T * H)
    seq2d = seq_lengths.astype(jnp.int32).reshape(B, 1)

    const = lambda i: (0, 0)
    xf_spec = pl.BlockSpec((B, t_chunk * H), lambda i: (0, i))
    xb_spec = pl.BlockSpec((B, t_chunk * H), lambda i: (0, n_chunks - 1 - i))
    wspec = lambda a: pl.BlockSpec(a.shape, const)

    kernel_fn = functools.partial(_bigru_chunk_kernel, t_chunk=t_chunk,
                                  unroll=min(unroll, t_chunk))

    blk_bytes = t_chunk * B * H * 4
    vmem_bytes = int(min(4 * 2 * blk_bytes
                         + 4 * H * 3 * H * 2
                         + 2 * B * H * 4 + (16 << 20),
                         56 << 20))

    out = pl.pallas_call(
        kernel_fn,
        out_shape=jax.ShapeDtypeStruct((T, B, 2 * H), jnp.float32),
        grid=(n_chunks,),
        in_specs=[
            pl.BlockSpec(seq2d.shape, const),
            xf_spec, xb_spec,
            wspec(wih_f), wspec(whh_f), wspec(brz_f), wspec(bihn_f),
            wspec(bhhn_f),
            wspec(wih_b), wspec(whh_b), wspec(brz_b), wspec(bihn_b),
            wspec(bhhn_b),
        ],
        out_specs=pl.BlockSpec(memory_space=pl.ANY),
        scratch_shapes=[
            pltpu.VMEM((B, H), jnp.float32),
            pltpu.VMEM((B, H), jnp.float32),
            pltpu.VMEM((2, t_chunk, B, H), jnp.float32),
            pltpu.VMEM((2, t_chunk, B, H), jnp.float32),
            pltpu.SemaphoreType.DMA((2,)),
            pltpu.SemaphoreType.DMA((2,)),
        ],
        compiler_params=pltpu.CompilerParams(
            dimension_semantics=("arbitrary",),
            vmem_limit_bytes=vmem_bytes),
    )(seq2d, x2v, x2v, wih_f, whh_f, brz_f, bihn_f, bhhn_f,
      wih_b, whh_b, brz_b, bihn_b, bhhn_b)

    return out


def kernel(x_bth, seq_lengths, w_ih_f, w_hh_f, b_ih_f, b_hh_f,
           w_ih_b, w_hh_b, b_ih_b, b_hh_b, embedding, fc_w, fc_b):
    H = x_bth.shape[-1]
    scale = jnp.concatenate([jnp.full((1, 2 * H), 0.5, jnp.float32),
                             jnp.ones((1, H), jnp.float32)], axis=-1)
    prep_w = lambda w: (w * scale).astype(jnp.bfloat16)
    prep_rz = lambda bi, bh: (bi + bh)[:, :2 * H] * jnp.float32(0.5)
    return _bigru(
        x_bth, seq_lengths,
        prep_w(w_ih_f), prep_w(w_hh_f), prep_rz(b_ih_f, b_hh_f),
        b_ih_f[:, 2 * H:], b_hh_f[:, 2 * H:],
        prep_w(w_ih_b), prep_w(w_hh_b), prep_rz(b_ih_b, b_hh_b),
        b_ih_b[:, 2 * H:], b_hh_b[:, 2 * H:])
